# baseline (speedup 1.0000x reference)
import functools
import math

import jax
import jax.numpy as jnp
from jax import lax
from jax.experimental import pallas as pl
from jax.experimental.pallas import tpu as pltpu
from jax.experimental.pallas import tpu_sc as plsc

NUM_CLASSES = 80
ANCHORS = [[1.3221, 1.73145], [3.19275, 4.00944], [5.05587, 8.09892],
           [9.47112, 4.84053], [11.2364, 10.0071]]
LAMBDA_OBJ = 10.0
LAMBDA_NOOBJ = 1.0
LAMBDA_COORD = 0.05
LAMBDA_CLASS = 1.0
IGNORE_THRESHOLD = 0.5

B, M, A, C, H, W = 128, 12, 5, 80, 26, 26
HW = H * W
CH = A * (5 + C)
NCELL = B * A * HW
OBJ_N = 85 * M * B
IGN_N = A * M * B
NW = 32
OBJ_PW = OBJ_N // NW
IGN_PW = IGN_N // NW


def _atan_pos(x):
    c = x > 2.414213562373095
    m = x > 0.4142135623730951
    t = jnp.where(c, -1.0 / x, jnp.where(m, (x - 1.0) / (x + 1.0), x))
    y0 = jnp.where(c, math.pi / 2, jnp.where(m, math.pi / 4, 0.0))
    z = t * t
    y = (((8.05374449538e-2 * z - 1.38776856032e-1) * z
          + 1.99777106478e-1) * z - 3.33329491539e-1) * z * t + t
    return y0 + y


def _bce(p, t):
    logp = jnp.maximum(jnp.log(jnp.maximum(p, 1e-43)), -100.0)
    log1mp = jnp.maximum(jnp.log(jnp.maximum(1.0 - p, 1e-43)), -100.0)
    return -(t * logp + (1.0 - t) * log1mp)


def _encode_body(tgt_ref, io_ref, ii_ref, lw_ref, cc_ref, cid_ref, tbox_ref,
                 icnt_ref):
    tgt = tgt_ref[...]
    cx = tgt[0]
    cy = tgt[1]
    w = tgt[2]
    h = tgt[3]
    cid = tgt[4]
    valid = (cx + cy + w + h + cid) > 0.0
    gx = cx * float(W)
    gy = cy * float(H)
    gw = w * float(W)
    gh = h * float(H)
    gi = gx.astype(jnp.int32)
    gj = gy.astype(jnp.int32)
    pos = gj * W + gi
    ious = []
    for a in range(A):
        aw, ah = ANCHORS[a]
        inter = jnp.minimum(gw, aw) * jnp.minimum(gh, ah)
        union = gw * gh + aw * ah - inter + 1e-7
        ious.append(inter / union)
    best = jnp.zeros(cx.shape, jnp.int32)
    bestv = ious[0]
    for a in range(1, A):
        cnd = ious[a] > bestv
        best = jnp.where(cnd, jnp.int32(a), best)
        bestv = jnp.maximum(bestv, ious[a])
    aw_b = jnp.zeros_like(gw)
    ah_b = jnp.zeros_like(gh)
    for a in range(A):
        sel = (best == a).astype(jnp.float32)
        aw_b = aw_b + sel * ANCHORS[a][0]
        ah_b = ah_b + sel * ANCHORS[a][1]
    tx = gx - gi.astype(jnp.float32)
    ty = gy - gj.astype(jnp.float32)
    tw = gw / aw_b
    th = gh / ah_b

    ti = lax.broadcasted_iota(jnp.int32, (M, M, 1), 0)
    tj = lax.broadcasted_iota(jnp.int32, (M, M, 1), 1)
    later = tj > ti
    earlier = tj < ti
    key = best * jnp.int32(HW) + pos
    keq = (key[:, None, :] == key[None, :, :]) & valid[None, :, :]
    lw = valid & jnp.logical_not(jnp.any(keq & later, axis=1))
    ceq = keq & (cid[:, None, :] == cid[None, :, :])
    cc = valid & jnp.logical_not(jnp.any(ceq & later, axis=1))
    poseq = pos[:, None, :] == pos[None, :, :]
    ep = earlier & poseq
    icnt_rows = []
    for a in range(A):
        ignf_a = valid & (ious[a] > IGNORE_THRESHOLD)
        conflict = jnp.any(ep & ignf_a[None, :, :], axis=1)
        icnt_rows.append(
            (ignf_a & jnp.logical_not(conflict)).astype(jnp.float32))
    icnt = jnp.stack(icnt_rows, axis=0)

    bi = lax.broadcasted_iota(jnp.int32, (M, B), 1)
    base_obj = bi * jnp.int32(CH * HW) + best * jnp.int32(85 * HW) + pos
    ci = lax.broadcasted_iota(jnp.int32, (85, M, B), 0)
    io_ref[...] = base_obj[None] + jnp.int32(HW) * ci
    ai = lax.broadcasted_iota(jnp.int32, (A, M, B), 0)
    ii_ref[...] = (bi[None] * jnp.int32(CH * HW)
                   + (jnp.int32(85) * ai + 4) * jnp.int32(HW) + pos[None])
    lw_ref[...] = lw.astype(jnp.float32)
    cc_ref[...] = cc.astype(jnp.float32)
    cid_ref[...] = cid
    tbox_ref[...] = jnp.stack([tx, ty, tw, th], axis=0)
    icnt_ref[...] = icnt


def _encode(tgt):
    f32 = jnp.float32
    return pl.pallas_call(
        _encode_body,
        out_shape=[
            jax.ShapeDtypeStruct((85, M, B), jnp.int32),
            jax.ShapeDtypeStruct((A, M, B), jnp.int32),
            jax.ShapeDtypeStruct((M, B), f32),
            jax.ShapeDtypeStruct((M, B), f32),
            jax.ShapeDtypeStruct((M, B), f32),
            jax.ShapeDtypeStruct((4, M, B), f32),
            jax.ShapeDtypeStruct((A, M, B), f32),
        ],
    )(tgt)


def _sc_gather(flat, idx_obj, idx_ign):
    mesh = plsc.VectorSubcoreMesh(core_axis_name="c", subcore_axis_name="s")

    @functools.partial(
        pl.kernel,
        mesh=mesh,
        out_type=[
            jax.ShapeDtypeStruct((OBJ_N,), jnp.float32),
            jax.ShapeDtypeStruct((IGN_N,), jnp.float32),
        ],
        scratch_types=[
            pltpu.VMEM((OBJ_PW,), jnp.int32),
            pltpu.VMEM((OBJ_PW,), jnp.float32),
            pltpu.VMEM((IGN_PW,), jnp.int32),
            pltpu.VMEM((IGN_PW,), jnp.float32),
            pltpu.SemaphoreType.DMA,
        ],
    )
    def k(flat_hbm, io_hbm, ii_hbm, out_obj, out_ign, io_v, vo_v, ii_v, vi_v,
          sem):
        wid = lax.axis_index("s") * 2 + lax.axis_index("c")
        ob = wid * OBJ_PW
        ib = wid * IGN_PW
        pltpu.sync_copy(io_hbm.at[pl.ds(ob, OBJ_PW)], io_v)
        pltpu.async_copy(flat_hbm.at[io_v], vo_v, sem).wait()
        pltpu.sync_copy(vo_v, out_obj.at[pl.ds(ob, OBJ_PW)])
        pltpu.sync_copy(ii_hbm.at[pl.ds(ib, IGN_PW)], ii_v)
        pltpu.async_copy(flat_hbm.at[ii_v], vi_v, sem).wait()
        pltpu.sync_copy(vi_v, out_ign.at[pl.ds(ib, IGN_PW)])

    return k(flat, idx_obj, idx_ign)


def _dense_body(blk_ref, out_ref, acc_ref):
    step = pl.program_id(0)

    @pl.when(step == 0)
    def _init():
        acc_ref[0, 0] = 0.0

    planes = jnp.stack([blk_ref[0, 85 * a + 4] for a in range(A)])
    p = jax.nn.sigmoid(planes)
    acc_ref[0, 0] += jnp.sum(_bce(p, jnp.zeros_like(p)))

    @pl.when(step == B - 1)
    def _fin():
        out_ref[0, 0] = acc_ref[0, 0]


def _dense(inp4):
    return pl.pallas_call(
        _dense_body,
        grid=(B,),
        in_specs=[pl.BlockSpec((1, CH, H, W), lambda b: (b, 0, 0, 0))],
        out_specs=pl.BlockSpec(memory_space=pltpu.SMEM),
        out_shape=jax.ShapeDtypeStruct((1, 1), jnp.float32),
        scratch_shapes=[pltpu.SMEM((1, 1), jnp.float32)],
    )(inp4)


def _records_body(dense_ref, vo_ref, vi_ref, lw_ref, cc_ref, cid_ref,
                  tbox_ref, icnt_ref, out_ref):
    if True:
        vo = vo_ref[...]
        lwf = lw_ref[...]
        ccf = cc_ref[...]
        cid = cid_ref[...]
        tbox = tbox_ref[...]
        icf = icnt_ref[...]

        pconf = jax.nn.sigmoid(vo[4])
        pcls = jax.nn.sigmoid(vo[5:])

        eps = 1e-7
        b1x = jax.nn.sigmoid(vo[0])
        b1y = jax.nn.sigmoid(vo[1])
        b1w = jnp.exp(vo[2])
        b1h = jnp.exp(vo[3])
        b2x, b2y, b2w, b2h = tbox[0], tbox[1], tbox[2], tbox[3]
        b1x1, b1x2 = b1x - b1w * 0.5, b1x + b1w * 0.5
        b1y1, b1y2 = b1y - b1h * 0.5, b1y + b1h * 0.5
        b2x1, b2x2 = b2x - b2w * 0.5, b2x + b2w * 0.5
        b2y1, b2y2 = b2y - b2h * 0.5, b2y + b2h * 0.5
        iw = jnp.clip(jnp.minimum(b1x2, b2x2) - jnp.maximum(b1x1, b2x1), 0.0)
        ih = jnp.clip(jnp.minimum(b1y2, b2y2) - jnp.maximum(b1y1, b2y1), 0.0)
        inter = iw * ih
        w1, h1 = b1x2 - b1x1, b1y2 - b1y1 + eps
        w2, h2 = b2x2 - b2x1, b2y2 - b2y1 + eps
        union = w1 * h1 + w2 * h2 - inter + eps
        iou = inter / union
        cw = jnp.maximum(b1x2, b2x2) - jnp.minimum(b1x1, b2x1)
        chh = jnp.maximum(b1y2, b2y2) - jnp.minimum(b1y1, b2y1)
        c2 = cw * cw + chh * chh + eps
        rho2 = ((b2x1 + b2x2 - b1x1 - b1x2) ** 2
                + (b2y1 + b2y2 - b1y1 - b1y2) ** 2) / 4.0
        v = (4.0 / math.pi ** 2) * (_atan_pos(w2 / h2)
                                    - _atan_pos(w1 / h1)) ** 2
        alpha = v / (v - iou + (1.0 + eps))
        ciou = iou - (rho2 / c2 + v * alpha)

        posm = (lwf > 0.0) & (ciou > 0.0)
        n_pos = jnp.maximum(jnp.sum(posm.astype(jnp.float32)), 1.0)
        box_loss = LAMBDA_COORD * jnp.sum(
            jnp.where(posm, 1.0 - ciou, 0.0)) / n_pos

        obj_sum = jnp.sum(lwf * _bce(pconf, jnp.ones_like(pconf)))
        object_loss = LAMBDA_OBJ * obj_sum / float(NCELL)

        pin = jax.nn.sigmoid(vi_ref[...])
        ign_sum = jnp.sum(icf * _bce(pin, jnp.zeros_like(pin)))
        no_object_loss = LAMBDA_NOOBJ * (dense_ref[0, 0] - ign_sum) / float(NCELL)

        cls_base = jnp.sum(lwf[None] * _bce(pcls, jnp.zeros_like(pcls)))
        onehot = (lax.broadcasted_iota(jnp.int32, (C, M, B), 0)
                  == cid[None].astype(jnp.int32)).astype(jnp.float32)
        dcorr = _bce(pcls, jnp.ones_like(pcls)) - _bce(pcls,
                                                       jnp.zeros_like(pcls))
        cls_corr = jnp.sum(ccf * jnp.sum(onehot * dcorr, axis=0))
        n_sel = jnp.maximum(jnp.sum(lwf) * float(C), 1.0)
        class_loss = LAMBDA_CLASS * (cls_base + cls_corr) / n_sel

        out_ref[0, 0] = (box_loss + object_loss + no_object_loss
                         + class_loss) * float(B)


def _records(dense, vo, vi, lw, cc, cid, tbox, icnt):
    smem = pl.BlockSpec(memory_space=pltpu.SMEM)
    return pl.pallas_call(
        _records_body,
        in_specs=[smem] + [pl.BlockSpec()] * 7,
        out_specs=smem,
        out_shape=jax.ShapeDtypeStruct((1, 1), jnp.float32),
    )(dense, vo, vi, lw, cc, cid, tbox, icnt)


def kernel(input, target):
    input = jnp.asarray(input, jnp.float32)
    target = jnp.asarray(target, jnp.float32)
    tgt = jnp.transpose(target[:, :M, :], (2, 1, 0))
    io, ii, lw, cc, cid, tbox, icnt = _encode(tgt)
    vo, vi = _sc_gather(input.reshape(-1), io.reshape(-1), ii.reshape(-1))
    dense = _dense(input)
    out = _records(dense, vo.reshape(85, M, B), vi.reshape(A, M, B),
                   lw, cc, cid, tbox, icnt)
    return out.reshape(())


# speedup vs baseline: 1.4489x; 1.4489x over previous
import functools
import math

import jax
import jax.numpy as jnp
from jax import lax
from jax.experimental import pallas as pl
from jax.experimental.pallas import tpu as pltpu
from jax.experimental.pallas import tpu_sc as plsc

NUM_CLASSES = 80
ANCHORS = [[1.3221, 1.73145], [3.19275, 4.00944], [5.05587, 8.09892],
           [9.47112, 4.84053], [11.2364, 10.0071]]
LAMBDA_OBJ = 10.0
LAMBDA_NOOBJ = 1.0
LAMBDA_COORD = 0.05
LAMBDA_CLASS = 1.0
IGNORE_THRESHOLD = 0.5

B, M, A, C, H, W = 128, 12, 5, 80, 26, 26
HW = H * W
CH = A * (5 + C)
NCELL = B * A * HW
OBJ_N = 85 * M * B
IGN_N = A * M * B
NW = 32
OBJ_PW = OBJ_N // NW
IGN_PW = IGN_N // NW


def _atan_pos(x):
    c = x > 2.414213562373095
    m = x > 0.4142135623730951
    t = jnp.where(c, -1.0 / x, jnp.where(m, (x - 1.0) / (x + 1.0), x))
    y0 = jnp.where(c, math.pi / 2, jnp.where(m, math.pi / 4, 0.0))
    z = t * t
    y = (((8.05374449538e-2 * z - 1.38776856032e-1) * z
          + 1.99777106478e-1) * z - 3.33329491539e-1) * z * t + t
    return y0 + y


def _bce(p, t):
    logp = jnp.maximum(jnp.log(jnp.maximum(p, 1e-43)), -100.0)
    log1mp = jnp.maximum(jnp.log(jnp.maximum(1.0 - p, 1e-43)), -100.0)
    return -(t * logp + (1.0 - t) * log1mp)


def _encode_body(tgt_ref, io_ref, ii_ref, lw_ref, cc_ref, cid_ref, tbox_ref,
                 icnt_ref):
    tgt = tgt_ref[...]
    cx = tgt[0]
    cy = tgt[1]
    w = tgt[2]
    h = tgt[3]
    cid = tgt[4]
    valid = (cx + cy + w + h + cid) > 0.0
    gx = cx * float(W)
    gy = cy * float(H)
    gw = w * float(W)
    gh = h * float(H)
    gi = gx.astype(jnp.int32)
    gj = gy.astype(jnp.int32)
    pos = gj * W + gi
    ious = []
    for a in range(A):
        aw, ah = ANCHORS[a]
        inter = jnp.minimum(gw, aw) * jnp.minimum(gh, ah)
        union = gw * gh + aw * ah - inter + 1e-7
        ious.append(inter / union)
    best = jnp.zeros(cx.shape, jnp.int32)
    bestv = ious[0]
    for a in range(1, A):
        cnd = ious[a] > bestv
        best = jnp.where(cnd, jnp.int32(a), best)
        bestv = jnp.maximum(bestv, ious[a])
    aw_b = jnp.zeros_like(gw)
    ah_b = jnp.zeros_like(gh)
    for a in range(A):
        sel = (best == a).astype(jnp.float32)
        aw_b = aw_b + sel * ANCHORS[a][0]
        ah_b = ah_b + sel * ANCHORS[a][1]
    tx = gx - gi.astype(jnp.float32)
    ty = gy - gj.astype(jnp.float32)
    tw = gw / aw_b
    th = gh / ah_b

    ti = lax.broadcasted_iota(jnp.int32, (M, M, 1), 0)
    tj = lax.broadcasted_iota(jnp.int32, (M, M, 1), 1)
    later = tj > ti
    earlier = tj < ti
    key = best * jnp.int32(HW) + pos
    keq = (key[:, None, :] == key[None, :, :]) & valid[None, :, :]
    lw = valid & jnp.logical_not(jnp.any(keq & later, axis=1))
    ceq = keq & (cid[:, None, :] == cid[None, :, :])
    cc = valid & jnp.logical_not(jnp.any(ceq & later, axis=1))
    poseq = pos[:, None, :] == pos[None, :, :]
    ep = earlier & poseq
    icnt_rows = []
    for a in range(A):
        ignf_a = valid & (ious[a] > IGNORE_THRESHOLD)
        conflict = jnp.any(ep & ignf_a[None, :, :], axis=1)
        icnt_rows.append(
            (ignf_a & jnp.logical_not(conflict)).astype(jnp.float32))
    icnt = jnp.stack(icnt_rows, axis=0)

    bi = lax.broadcasted_iota(jnp.int32, (M, B), 1)
    base_obj = bi * jnp.int32(CH * HW) + best * jnp.int32(85 * HW) + pos
    ci = lax.broadcasted_iota(jnp.int32, (85, M, B), 0)
    io_ref[...] = base_obj[None] + jnp.int32(HW) * ci
    ai = lax.broadcasted_iota(jnp.int32, (A, M, B), 0)
    ii_ref[...] = (bi[None] * jnp.int32(CH * HW)
                   + (jnp.int32(85) * ai + 4) * jnp.int32(HW) + pos[None])
    lw_ref[...] = lw.astype(jnp.float32)
    cc_ref[...] = cc.astype(jnp.float32)
    cid_ref[...] = cid
    tbox_ref[...] = jnp.stack([tx, ty, tw, th], axis=0)
    icnt_ref[...] = icnt


def _encode(tgt):
    f32 = jnp.float32
    return pl.pallas_call(
        _encode_body,
        out_shape=[
            jax.ShapeDtypeStruct((85, M, B), jnp.int32),
            jax.ShapeDtypeStruct((A, M, B), jnp.int32),
            jax.ShapeDtypeStruct((M, B), f32),
            jax.ShapeDtypeStruct((M, B), f32),
            jax.ShapeDtypeStruct((M, B), f32),
            jax.ShapeDtypeStruct((4, M, B), f32),
            jax.ShapeDtypeStruct((A, M, B), f32),
        ],
    )(tgt)


def _sc_gather(flat, idx_obj, idx_ign):
    mesh = plsc.VectorSubcoreMesh(core_axis_name="c", subcore_axis_name="s")

    @functools.partial(
        pl.kernel,
        mesh=mesh,
        out_type=[
            jax.ShapeDtypeStruct((OBJ_N,), jnp.float32),
            jax.ShapeDtypeStruct((IGN_N,), jnp.float32),
        ],
        scratch_types=[
            pltpu.VMEM((OBJ_PW,), jnp.int32),
            pltpu.VMEM((OBJ_PW,), jnp.float32),
            pltpu.VMEM((IGN_PW,), jnp.int32),
            pltpu.VMEM((IGN_PW,), jnp.float32),
            pltpu.SemaphoreType.DMA,
        ],
    )
    def k(flat_hbm, io_hbm, ii_hbm, out_obj, out_ign, io_v, vo_v, ii_v, vi_v,
          sem):
        wid = lax.axis_index("s") * 2 + lax.axis_index("c")
        ob = wid * OBJ_PW
        ib = wid * IGN_PW
        pltpu.sync_copy(io_hbm.at[pl.ds(ob, OBJ_PW)], io_v)
        pltpu.async_copy(flat_hbm.at[io_v], vo_v, sem).wait()
        pltpu.sync_copy(vo_v, out_obj.at[pl.ds(ob, OBJ_PW)])
        pltpu.sync_copy(ii_hbm.at[pl.ds(ib, IGN_PW)], ii_v)
        pltpu.async_copy(flat_hbm.at[ii_v], vi_v, sem).wait()
        pltpu.sync_copy(vi_v, out_ign.at[pl.ds(ib, IGN_PW)])

    return k(flat, idx_obj, idx_ign)


def _loss_body(conf_ref, vo_ref, vi_ref, lw_ref, cc_ref, cid_ref, tbox_ref,
               icnt_ref, out_ref, acc_ref):
    step = pl.program_id(0)

    @pl.when(step == 0)
    def _init():
        acc_ref[0, 0] = 0.0

    p = jax.nn.sigmoid(conf_ref[...])
    acc_ref[0, 0] += jnp.sum(_bce(p, jnp.zeros_like(p)))

    @pl.when(step == A - 1)
    def _finish():
        vo = vo_ref[...]
        lwf = lw_ref[...]
        ccf = cc_ref[...]
        cid = cid_ref[...]
        tbox = tbox_ref[...]
        icf = icnt_ref[...]

        pconf = jax.nn.sigmoid(vo[4])
        pcls = jax.nn.sigmoid(vo[5:])

        eps = 1e-7
        b1x = jax.nn.sigmoid(vo[0])
        b1y = jax.nn.sigmoid(vo[1])
        b1w = jnp.exp(vo[2])
        b1h = jnp.exp(vo[3])
        b2x, b2y, b2w, b2h = tbox[0], tbox[1], tbox[2], tbox[3]
        b1x1, b1x2 = b1x - b1w * 0.5, b1x + b1w * 0.5
        b1y1, b1y2 = b1y - b1h * 0.5, b1y + b1h * 0.5
        b2x1, b2x2 = b2x - b2w * 0.5, b2x + b2w * 0.5
        b2y1, b2y2 = b2y - b2h * 0.5, b2y + b2h * 0.5
        iw = jnp.clip(jnp.minimum(b1x2, b2x2) - jnp.maximum(b1x1, b2x1), 0.0)
        ih = jnp.clip(jnp.minimum(b1y2, b2y2) - jnp.maximum(b1y1, b2y1), 0.0)
        inter = iw * ih
        w1, h1 = b1x2 - b1x1, b1y2 - b1y1 + eps
        w2, h2 = b2x2 - b2x1, b2y2 - b2y1 + eps
        union = w1 * h1 + w2 * h2 - inter + eps
        iou = inter / union
        cw = jnp.maximum(b1x2, b2x2) - jnp.minimum(b1x1, b2x1)
        chh = jnp.maximum(b1y2, b2y2) - jnp.minimum(b1y1, b2y1)
        c2 = cw * cw + chh * chh + eps
        rho2 = ((b2x1 + b2x2 - b1x1 - b1x2) ** 2
                + (b2y1 + b2y2 - b1y1 - b1y2) ** 2) / 4.0
        v = (4.0 / math.pi ** 2) * (_atan_pos(w2 / h2)
                                    - _atan_pos(w1 / h1)) ** 2
        alpha = v / (v - iou + (1.0 + eps))
        ciou = iou - (rho2 / c2 + v * alpha)

        posm = (lwf > 0.0) & (ciou > 0.0)
        n_pos = jnp.maximum(jnp.sum(posm.astype(jnp.float32)), 1.0)
        box_loss = LAMBDA_COORD * jnp.sum(
            jnp.where(posm, 1.0 - ciou, 0.0)) / n_pos

        obj_sum = jnp.sum(lwf * _bce(pconf, jnp.ones_like(pconf)))
        object_loss = LAMBDA_OBJ * obj_sum / float(NCELL)

        pin = jax.nn.sigmoid(vi_ref[...])
        ign_sum = jnp.sum(icf * _bce(pin, jnp.zeros_like(pin)))
        no_object_loss = LAMBDA_NOOBJ * (acc_ref[0, 0] - ign_sum) / float(NCELL)

        cls_base = jnp.sum(lwf[None] * _bce(pcls, jnp.zeros_like(pcls)))
        onehot = (lax.broadcasted_iota(jnp.int32, (C, M, B), 0)
                  == cid[None].astype(jnp.int32)).astype(jnp.float32)
        dcorr = _bce(pcls, jnp.ones_like(pcls)) - _bce(pcls,
                                                       jnp.zeros_like(pcls))
        cls_corr = jnp.sum(ccf * jnp.sum(onehot * dcorr, axis=0))
        n_sel = jnp.maximum(jnp.sum(lwf) * float(C), 1.0)
        class_loss = LAMBDA_CLASS * (cls_base + cls_corr) / n_sel

        out_ref[0, 0] = (box_loss + object_loss + no_object_loss
                         + class_loss) * float(B)


def _loss(inp4, vo, vi, lw, cc, cid, tbox, icnt):
    whole = lambda shape: pl.BlockSpec(shape, lambda a: (0,) * len(shape))
    return pl.pallas_call(
        _loss_body,
        grid=(A,),
        in_specs=[
            pl.BlockSpec((B, 1, 1, HW), lambda a: (0, 85 * a + 4, 0, 0)),
            whole((85, M, B)),
            whole((A, M, B)),
            whole((M, B)),
            whole((M, B)),
            whole((M, B)),
            whole((4, M, B)),
            whole((A, M, B)),
        ],
        out_specs=pl.BlockSpec(memory_space=pltpu.SMEM),
        out_shape=jax.ShapeDtypeStruct((1, 1), jnp.float32),
        scratch_shapes=[pltpu.SMEM((1, 1), jnp.float32)],
    )(inp4, vo, vi, lw, cc, cid, tbox, icnt)


def kernel(input, target):
    input = jnp.asarray(input, jnp.float32)
    target = jnp.asarray(target, jnp.float32)
    tgt = jnp.transpose(target[:, :M, :], (2, 1, 0))
    io, ii, lw, cc, cid, tbox, icnt = _encode(tgt)
    vo, vi = _sc_gather(input.reshape(-1), io.reshape(-1), ii.reshape(-1))
    out = _loss(input.reshape(B, CH, 1, HW), vo.reshape(85, M, B),
                vi.reshape(A, M, B), lw, cc, cid, tbox, icnt)
    return out.reshape(())
